# baseline (speedup 1.0000x reference)
import jax
import jax.numpy as jnp
from jax import lax
from jax.experimental import pallas as pl
from jax.experimental.pallas import tpu as pltpu
from jax.experimental.pallas import tpu_sc as plsc

_SEQ = 1025
_HEADS = 16
_DIST = 3972
_NW = 32
_FULL = 64
_TAIL = _SEQ - 16
_N_CHUNKS = 4


def _make_body(row0, nrows):
    def body(table_t_hbm, idx_hbm, out_hbm, table_v,
             idx_v0, idx_v1, out_v0, out_v1, sem_idx, sem_out):
        cid = lax.axis_index("c")
        sid = lax.axis_index("s")
        wid = sid * 2 + cid
        pltpu.sync_copy(table_t_hbm, table_v)

        idx_bufs = (idx_v0, idx_v1)
        out_bufs = (out_v0, out_v1)

        def idx_copy(k, buf):
            return pltpu.make_async_copy(
                idx_hbm.at[pl.ds(row0 + k, 1), :], buf, sem_idx)

        def out_copy(k, buf):
            return pltpu.make_async_copy(
                buf, out_hbm.at[:, pl.ds(k, 1), :], sem_out)

        def compute_row(idx_v, out_v):
            def gather_vec(start, carry):
                iv = idx_v[0, pl.ds(start, 16)]
                vals = [plsc.load_gather(table_v, [iv + (h * _DIST)])
                        for h in range(_HEADS)]
                for h in range(_HEADS):
                    out_v[h, 0, pl.ds(start, 16)] = vals[h]
                return carry

            lax.fori_loop(0, _FULL, lambda c, k: gather_vec(c * 16, k), 0,
                          unroll=2)
            gather_vec(_TAIL, 0)

        idx_copy(wid, idx_bufs[0]).start()

        n_pairs = (nrows + 2 * _NW - 1) // (2 * _NW)

        def pair(i2, carry):
            for b in range(2):
                i = 2 * i2 + b
                k = wid + _NW * i

                @pl.when(k < nrows)
                def _():
                    idx_copy(k, idx_bufs[b]).wait()

                    @pl.when(k + _NW < nrows)
                    def _():
                        idx_copy(k + _NW, idx_bufs[1 - b]).start()

                    @pl.when(i2 >= 1)
                    def _():
                        out_copy(k, out_bufs[b]).wait()

                    compute_row(idx_bufs[b], out_bufs[b])
                    out_copy(k, out_bufs[b]).start()

            return carry

        lax.fori_loop(0, n_pairs, pair, 0)

        out_copy(wid, out_v0).wait()
        out_copy(wid, out_v1).wait()

    return body


def kernel(relative_position_bias_table, relative_position_index):
    table_t = relative_position_bias_table.T.reshape(-1)
    mesh = plsc.VectorSubcoreMesh(core_axis_name="c", subcore_axis_name="s")

    bounds = [(_SEQ * c) // _N_CHUNKS for c in range(_N_CHUNKS + 1)]
    parts = []
    for c in range(_N_CHUNKS):
        row0, row1 = bounds[c], bounds[c + 1]
        nrows = row1 - row0
        run = pl.kernel(
            _make_body(row0, nrows),
            out_type=jax.ShapeDtypeStruct((_HEADS, nrows, _SEQ), jnp.float32),
            mesh=mesh,
            scratch_types=[
                pltpu.VMEM((_HEADS * _DIST,), jnp.float32),
                pltpu.VMEM((1, _SEQ), jnp.int32),
                pltpu.VMEM((1, _SEQ), jnp.int32),
                pltpu.VMEM((_HEADS, 1, _SEQ), jnp.float32),
                pltpu.VMEM((_HEADS, 1, _SEQ), jnp.float32),
                pltpu.SemaphoreType.DMA,
                pltpu.SemaphoreType.DMA,
            ],
            compiler_params=pltpu.CompilerParams(needs_layout_passes=False),
        )
        parts.append(run(table_t, relative_position_index))

    return jnp.concatenate(parts, axis=1)


# speedup vs baseline: 3.0138x; 3.0138x over previous
import jax
import jax.numpy as jnp
from jax import lax
from jax.experimental import pallas as pl
from jax.experimental.pallas import tpu as pltpu
from jax.experimental.pallas import tpu_sc as plsc

_SEQ = 1025
_HEADS = 16
_DIST = 3972
_NW = 32
_FULL = 64
_TAIL = _SEQ - 16


def _sc_body(table_t_hbm, idx_hbm, out_hbm, table_v,
             idx_v0, idx_v1, out_v0, out_v1, sem_idx, sem_out):
    cid = lax.axis_index("c")
    sid = lax.axis_index("s")
    wid = sid * 2 + cid
    pltpu.sync_copy(table_t_hbm, table_v)

    idx_bufs = (idx_v0, idx_v1)
    out_bufs = (out_v0, out_v1)

    def idx_copy(r, buf):
        return pltpu.make_async_copy(idx_hbm.at[pl.ds(r, 1), :], buf, sem_idx)

    def out_copy(r, buf):
        return pltpu.make_async_copy(buf, out_hbm.at[pl.ds(r, 1), :, :], sem_out)

    def compute_row(idx_v, out_v):
        def gather_vec(start, carry):
            iv = idx_v[0, pl.ds(start, 16)]
            vals = [plsc.load_gather(table_v, [iv + (h * _DIST)])
                    for h in range(_HEADS)]
            for h in range(_HEADS):
                out_v[0, h, pl.ds(start, 16)] = vals[h]
            return carry

        lax.fori_loop(0, _FULL, lambda c, k: gather_vec(c * 16, k), 0,
                      unroll=2)
        gather_vec(_TAIL, 0)

    idx_copy(wid, idx_v0).start()

    def pair(i2, carry):
        for b in range(2):
            i = 2 * i2 + b
            r = wid + _NW * i

            @pl.when(r < _SEQ)
            def _():
                idx_copy(r, idx_bufs[b]).wait()

                @pl.when(r + _NW < _SEQ)
                def _():
                    idx_copy(r + _NW, idx_bufs[1 - b]).start()

                @pl.when(i2 >= 1)
                def _():
                    out_copy(r, out_bufs[b]).wait()

                compute_row(idx_bufs[b], out_bufs[b])
                out_copy(r, out_bufs[b]).start()

        return carry

    lax.fori_loop(0, 17, pair, 0)

    out_copy(wid, out_v0).wait()
    out_copy(wid, out_v1).wait()


def kernel(relative_position_bias_table, relative_position_index):
    table_t = relative_position_bias_table.T.reshape(-1)
    mesh = plsc.VectorSubcoreMesh(core_axis_name="c", subcore_axis_name="s")
    run = pl.kernel(
        _sc_body,
        out_type=jax.ShapeDtypeStruct((_SEQ, _HEADS, _SEQ), jnp.float32),
        mesh=mesh,
        scratch_types=[
            pltpu.VMEM((_HEADS * _DIST,), jnp.float32),
            pltpu.VMEM((1, _SEQ), jnp.int32),
            pltpu.VMEM((1, _SEQ), jnp.int32),
            pltpu.VMEM((1, _HEADS, _SEQ), jnp.float32),
            pltpu.VMEM((1, _HEADS, _SEQ), jnp.float32),
            pltpu.SemaphoreType.DMA,
            pltpu.SemaphoreType.DMA,
        ],
        compiler_params=pltpu.CompilerParams(needs_layout_passes=False),
    )
    out_t = run(table_t, relative_position_index)
    return jnp.transpose(out_t, (1, 0, 2))
